# baseline (speedup 1.0000x reference)
import functools

import jax
import jax.numpy as jnp
from jax import lax
from jax.experimental import pallas as pl
from jax.experimental.pallas import tpu as pltpu
from jax.experimental.pallas import tpu_sc as plsc

VOCAB = 100000
EMB = 256
D1 = 256
D2 = 128


_BLK = 2000


def _mlp_body(t_ref, w1_ref, b1_ref, w2_ref, b2_ref, o_ref):
    h = jnp.dot(t_ref[...], w1_ref[...], preferred_element_type=jnp.float32)
    h = jnp.maximum(h + b1_ref[...], 0.0)
    o = jnp.dot(h, w2_ref[...], preferred_element_type=jnp.float32)
    o_ref[...] = jnp.maximum(o + b2_ref[...], 0.0)


def _fuse_table(table, W1, b1, W2, b2):
    grid = (VOCAB // _BLK,)
    return pl.pallas_call(
        _mlp_body,
        grid=grid,
        in_specs=[
            pl.BlockSpec((_BLK, EMB), lambda i: (i, 0)),
            pl.BlockSpec((EMB, D1), lambda i: (0, 0)),
            pl.BlockSpec((1, D1), lambda i: (0, 0)),
            pl.BlockSpec((D1, D2), lambda i: (0, 0)),
            pl.BlockSpec((1, D2), lambda i: (0, 0)),
        ],
        out_specs=pl.BlockSpec((_BLK, D2), lambda i: (i, 0)),
        out_shape=jax.ShapeDtypeStruct((VOCAB, D2), jnp.float32),
    )(table, W1, b1.reshape(1, D1), W2, b2.reshape(1, D2))


_NC = 2
_NS = 16
_NW = _NC * _NS
_CH = 128
_NB = 5


def _make_gather(n_tokens):
    assert n_tokens % (_NW * _CH) == 0
    n_ch = n_tokens // (_NW * _CH)
    b_per_w = n_ch * _CH
    mesh = plsc.VectorSubcoreMesh(core_axis_name="c", subcore_axis_name="s")

    @functools.partial(
        pl.kernel,
        out_type=jax.ShapeDtypeStruct((n_tokens, D2), jnp.float32),
        mesh=mesh,
        scratch_types=[
            pltpu.VMEM((n_ch, _CH), jnp.int32),
            pltpu.VMEM((_NB, _CH, D2), jnp.float32),
        ]
        + [pltpu.SemaphoreType.DMA] * (2 * _NB),
    )
    def gather_k(f_hbm, idx_hbm, out_hbm, idx_v, rows_v, *sems):
        gsem = sems[:_NB]
        ssem = sems[_NB:]
        wid = lax.axis_index("s") * _NC + lax.axis_index("c")
        pltpu.sync_copy(idx_hbm.at[wid], idx_v)
        base = wid * b_per_w

        def start_gather(j, b):
            pltpu.make_async_copy(f_hbm.at[idx_v.at[j]], rows_v.at[b], gsem[b]).start()

        def wait_gather(b):
            pltpu.make_async_copy(f_hbm.at[idx_v.at[0]], rows_v.at[b], gsem[b]).wait()

        def start_scatter(j, b):
            pltpu.make_async_copy(
                rows_v.at[b], out_hbm.at[pl.ds(base + j * _CH, _CH)], ssem[b]
            ).start()

        def wait_scatter(b):
            pltpu.make_async_copy(
                rows_v.at[b], out_hbm.at[pl.ds(base, _CH)], ssem[b]
            ).wait()

        start_gather(0, 0)

        @pl.loop(0, n_ch, step=_NB)
        def _(j):
            for b in range(_NB):
                jj = j + b
                wait_gather(b)
                start_scatter(jj, b)
                nxt = jj + 1
                bn = (b + 1) % _NB

                @pl.when(nxt < n_ch)
                def _():
                    @pl.when(nxt >= _NB)
                    def _():
                        wait_scatter(bn)

                    start_gather(nxt, bn)

        for b in range(_NB):
            wait_scatter(b)

    return gather_k


_gather_tokens = _make_gather(1024 * 200)


def kernel(inputs, table, W1, b1, W2, b2, is_training):
    fused = _fuse_table(table, W1, b1, W2, b2)
    b, t = inputs.shape
    idx3d = inputs.astype(jnp.int32).reshape(_NW, b * t // (_NW * _CH), _CH)
    out = _gather_tokens(fused, idx3d)
    return out.reshape(b, t, D2)


# speedup vs baseline: 1.1082x; 1.1082x over previous
import functools

import jax
import jax.numpy as jnp
from jax import lax
from jax.experimental import pallas as pl
from jax.experimental.pallas import tpu as pltpu
from jax.experimental.pallas import tpu_sc as plsc

VOCAB = 100000
EMB = 256
D1 = 256
D2 = 128


_BLK = 2000


def _mlp_body(t_ref, w1_ref, b1_ref, w2_ref, b2_ref, o_ref):
    h = jnp.dot(t_ref[...], w1_ref[...], preferred_element_type=jnp.float32)
    h = jnp.maximum(h + b1_ref[...], 0.0)
    o = jnp.dot(h, w2_ref[...], preferred_element_type=jnp.float32)
    o_ref[...] = jnp.maximum(o + b2_ref[...], 0.0)


def _fuse_table(table, W1, b1, W2, b2):
    grid = (VOCAB // _BLK,)
    return pl.pallas_call(
        _mlp_body,
        grid=grid,
        in_specs=[
            pl.BlockSpec((_BLK, EMB), lambda i: (i, 0)),
            pl.BlockSpec((EMB, D1), lambda i: (0, 0)),
            pl.BlockSpec((1, D1), lambda i: (0, 0)),
            pl.BlockSpec((D1, D2), lambda i: (0, 0)),
            pl.BlockSpec((1, D2), lambda i: (0, 0)),
        ],
        out_specs=pl.BlockSpec((_BLK, D2), lambda i: (i, 0)),
        out_shape=jax.ShapeDtypeStruct((VOCAB, D2), jnp.float32),
    )(table, W1, b1.reshape(1, D1), W2, b2.reshape(1, D2))


_NC = 2
_NS = 16
_NW = _NC * _NS
_CH = 128
_NB = 5
_PF = 2


def _make_gather(n_tokens):
    assert n_tokens % (_NW * _CH) == 0
    n_ch = n_tokens // (_NW * _CH)
    b_per_w = n_ch * _CH
    mesh = plsc.VectorSubcoreMesh(core_axis_name="c", subcore_axis_name="s")

    @functools.partial(
        pl.kernel,
        out_type=jax.ShapeDtypeStruct((n_tokens, D2), jnp.float32),
        mesh=mesh,
        scratch_types=[
            pltpu.VMEM((n_ch, _CH), jnp.int32),
            pltpu.VMEM((_NB, _CH, D2), jnp.float32),
        ]
        + [pltpu.SemaphoreType.DMA] * (2 * _NB),
    )
    def gather_k(f_hbm, idx_hbm, out_hbm, idx_v, rows_v, *sems):
        gsem = sems[:_NB]
        ssem = sems[_NB:]
        wid = lax.axis_index("s") * _NC + lax.axis_index("c")
        pltpu.sync_copy(idx_hbm.at[wid], idx_v)
        base = wid * b_per_w

        def start_gather(j, b):
            pltpu.make_async_copy(f_hbm.at[idx_v.at[j]], rows_v.at[b], gsem[b]).start()

        def wait_gather(b):
            pltpu.make_async_copy(f_hbm.at[idx_v.at[0]], rows_v.at[b], gsem[b]).wait()

        def start_scatter(j, b):
            pltpu.make_async_copy(
                rows_v.at[b], out_hbm.at[pl.ds(base + j * _CH, _CH)], ssem[b]
            ).start()

        def wait_scatter(b):
            pltpu.make_async_copy(
                rows_v.at[b], out_hbm.at[pl.ds(base, _CH)], ssem[b]
            ).wait()

        for p in range(_PF):
            start_gather(p, p)

        @pl.loop(0, n_ch, step=_NB)
        def _(j):
            for b in range(_NB):
                jj = j + b
                wait_gather(b)
                start_scatter(jj, b)
                nxt = jj + _PF
                bn = (b + _PF) % _NB

                @pl.when(nxt < n_ch)
                def _():
                    @pl.when(nxt >= _NB)
                    def _():
                        wait_scatter(bn)

                    start_gather(nxt, bn)

        for b in range(_NB):
            wait_scatter(b)

    return gather_k


_gather_tokens = _make_gather(1024 * 200)


def kernel(inputs, table, W1, b1, W2, b2, is_training):
    fused = _fuse_table(table, W1, b1, W2, b2)
    b, t = inputs.shape
    idx3d = inputs.astype(jnp.int32).reshape(_NW, b * t // (_NW * _CH), _CH)
    out = _gather_tokens(fused, idx3d)
    return out.reshape(b, t, D2)


# speedup vs baseline: 1.1108x; 1.0024x over previous
import functools

import jax
import jax.numpy as jnp
from jax import lax
from jax.experimental import pallas as pl
from jax.experimental.pallas import tpu as pltpu
from jax.experimental.pallas import tpu_sc as plsc

VOCAB = 100000
EMB = 256
D1 = 256
D2 = 128


_BLK = 2000


def _mlp_body(t_ref, w1_ref, b1_ref, w2_ref, b2_ref, o_ref):
    h = jnp.dot(t_ref[...], w1_ref[...], preferred_element_type=jnp.float32)
    h = jnp.maximum(h + b1_ref[...], 0.0)
    o = jnp.dot(h, w2_ref[...], preferred_element_type=jnp.float32)
    o_ref[...] = jnp.maximum(o + b2_ref[...], 0.0)


def _fuse_table(table, W1, b1, W2, b2):
    grid = (VOCAB // _BLK,)
    return pl.pallas_call(
        _mlp_body,
        grid=grid,
        in_specs=[
            pl.BlockSpec((_BLK, EMB), lambda i: (i, 0)),
            pl.BlockSpec((EMB, D1), lambda i: (0, 0)),
            pl.BlockSpec((1, D1), lambda i: (0, 0)),
            pl.BlockSpec((D1, D2), lambda i: (0, 0)),
            pl.BlockSpec((1, D2), lambda i: (0, 0)),
        ],
        out_specs=pl.BlockSpec((_BLK, D2), lambda i: (i, 0)),
        out_shape=jax.ShapeDtypeStruct((VOCAB, D2), jnp.float32),
    )(table, W1, b1.reshape(1, D1), W2, b2.reshape(1, D2))


_NC = 2
_NS = 16
_NW = _NC * _NS
_CH = 128
_NB = 5
_PF = 3


def _make_gather(n_tokens):
    assert n_tokens % (_NW * _CH) == 0
    n_ch = n_tokens // (_NW * _CH)
    b_per_w = n_ch * _CH
    mesh = plsc.VectorSubcoreMesh(core_axis_name="c", subcore_axis_name="s")

    @functools.partial(
        pl.kernel,
        out_type=jax.ShapeDtypeStruct((n_tokens, D2), jnp.float32),
        mesh=mesh,
        scratch_types=[
            pltpu.VMEM((n_ch, _CH), jnp.int32),
            pltpu.VMEM((_NB, _CH, D2), jnp.float32),
        ]
        + [pltpu.SemaphoreType.DMA] * (2 * _NB),
    )
    def gather_k(f_hbm, idx_hbm, out_hbm, idx_v, rows_v, *sems):
        gsem = sems[:_NB]
        ssem = sems[_NB:]
        wid = lax.axis_index("s") * _NC + lax.axis_index("c")
        pltpu.sync_copy(idx_hbm.at[wid], idx_v)
        base = wid * b_per_w

        def start_gather(j, b):
            pltpu.make_async_copy(f_hbm.at[idx_v.at[j]], rows_v.at[b], gsem[b]).start()

        def wait_gather(b):
            pltpu.make_async_copy(f_hbm.at[idx_v.at[0]], rows_v.at[b], gsem[b]).wait()

        def start_scatter(j, b):
            pltpu.make_async_copy(
                rows_v.at[b], out_hbm.at[pl.ds(base + j * _CH, _CH)], ssem[b]
            ).start()

        def wait_scatter(b):
            pltpu.make_async_copy(
                rows_v.at[b], out_hbm.at[pl.ds(base, _CH)], ssem[b]
            ).wait()

        for p in range(_PF):
            start_gather(p, p)

        @pl.loop(0, n_ch, step=_NB)
        def _(j):
            for b in range(_NB):
                jj = j + b
                wait_gather(b)
                start_scatter(jj, b)
                nxt = jj + _PF
                bn = (b + _PF) % _NB

                @pl.when(nxt < n_ch)
                def _():
                    @pl.when(nxt >= _NB)
                    def _():
                        wait_scatter(bn)

                    start_gather(nxt, bn)

        for b in range(_NB):
            wait_scatter(b)

    return gather_k


_gather_tokens = _make_gather(1024 * 200)


def kernel(inputs, table, W1, b1, W2, b2, is_training):
    fused = _fuse_table(table, W1, b1, W2, b2)
    b, t = inputs.shape
    idx3d = inputs.astype(jnp.int32).reshape(_NW, b * t // (_NW * _CH), _CH)
    out = _gather_tokens(fused, idx3d)
    return out.reshape(b, t, D2)


# speedup vs baseline: 1.2373x; 1.1138x over previous
import functools

import jax
import jax.numpy as jnp
from jax import lax
from jax.experimental import pallas as pl
from jax.experimental.pallas import tpu as pltpu
from jax.experimental.pallas import tpu_sc as plsc

VOCAB = 100000
EMB = 256
D1 = 256
D2 = 128


_BLK = 4000


def _mlp_body(t_ref, w1_ref, b1_ref, w2_ref, b2_ref, o_ref):
    h = jnp.dot(t_ref[...], w1_ref[...], preferred_element_type=jnp.float32)
    h = jnp.maximum(h + b1_ref[...], 0.0)
    o = jnp.dot(h, w2_ref[...], preferred_element_type=jnp.float32)
    o_ref[...] = jnp.maximum(o + b2_ref[...], 0.0)


def _fuse_table(table, W1, b1, W2, b2):
    grid = (VOCAB // _BLK,)
    return pl.pallas_call(
        _mlp_body,
        grid=grid,
        in_specs=[
            pl.BlockSpec((_BLK, EMB), lambda i: (i, 0)),
            pl.BlockSpec((EMB, D1), lambda i: (0, 0)),
            pl.BlockSpec((1, D1), lambda i: (0, 0)),
            pl.BlockSpec((D1, D2), lambda i: (0, 0)),
            pl.BlockSpec((1, D2), lambda i: (0, 0)),
        ],
        out_specs=pl.BlockSpec((_BLK, D2), lambda i: (i, 0)),
        out_shape=jax.ShapeDtypeStruct((VOCAB, D2), jnp.float32),
    )(table, W1, b1.reshape(1, D1), W2, b2.reshape(1, D2))


_NC = 2
_NS = 16
_NW = _NC * _NS
_CH = 128
_NB = 5
_PF = 3


def _make_gather(n_tokens):
    assert n_tokens % (_NW * _CH) == 0
    n_ch = n_tokens // (_NW * _CH)
    b_per_w = n_ch * _CH
    mesh = plsc.VectorSubcoreMesh(core_axis_name="c", subcore_axis_name="s")

    @functools.partial(
        pl.kernel,
        out_type=jax.ShapeDtypeStruct((n_tokens, D2), jnp.float32),
        mesh=mesh,
        scratch_types=[
            pltpu.VMEM((n_ch, _CH), jnp.int32),
            pltpu.VMEM((_NB, _CH, D2), jnp.float32),
        ]
        + [pltpu.SemaphoreType.DMA] * (2 * _NB),
    )
    def gather_k(f_hbm, idx_hbm, out_hbm, idx_v, rows_v, *sems):
        gsem = sems[:_NB]
        ssem = sems[_NB:]
        wid = lax.axis_index("s") * _NC + lax.axis_index("c")
        pltpu.sync_copy(idx_hbm.at[wid], idx_v)
        base = wid * b_per_w

        def start_gather(j, b):
            pltpu.make_async_copy(f_hbm.at[idx_v.at[j]], rows_v.at[b], gsem[b]).start()

        def wait_gather(b):
            pltpu.make_async_copy(f_hbm.at[idx_v.at[0]], rows_v.at[b], gsem[b]).wait()

        def start_scatter(j, b):
            pltpu.make_async_copy(
                rows_v.at[b], out_hbm.at[pl.ds(base + j * _CH, _CH)], ssem[b]
            ).start()

        def wait_scatter(b):
            pltpu.make_async_copy(
                rows_v.at[b], out_hbm.at[pl.ds(base, _CH)], ssem[b]
            ).wait()

        for p in range(_PF):
            start_gather(p, p)

        @pl.loop(0, n_ch, step=_NB)
        def _(j):
            for b in range(_NB):
                jj = j + b
                wait_gather(b)
                start_scatter(jj, b)
                nxt = jj + _PF
                bn = (b + _PF) % _NB

                @pl.when(nxt < n_ch)
                def _():
                    @pl.when(nxt >= _NB)
                    def _():
                        wait_scatter(bn)

                    start_gather(nxt, bn)

        for b in range(_NB):
            wait_scatter(b)

    return gather_k


_gather_tokens = _make_gather(1024 * 200)


def kernel(inputs, table, W1, b1, W2, b2, is_training):
    fused = _fuse_table(table, W1, b1, W2, b2)
    b, t = inputs.shape
    idx3d = inputs.astype(jnp.int32).reshape(_NW, b * t // (_NW * _CH), _CH)
    out = _gather_tokens(fused, idx3d)
    return out.reshape(b, t, D2)


# speedup vs baseline: 1.2816x; 1.0359x over previous
import functools

import jax
import jax.numpy as jnp
from jax import lax
from jax.experimental import pallas as pl
from jax.experimental.pallas import tpu as pltpu
from jax.experimental.pallas import tpu_sc as plsc

VOCAB = 100000
EMB = 256
D1 = 256
D2 = 128


_BLK = 10000


def _mlp_body(t_ref, w1_ref, b1_ref, w2_ref, b2_ref, o_ref):
    h = jnp.dot(t_ref[...], w1_ref[...], preferred_element_type=jnp.float32)
    h = jnp.maximum(h + b1_ref[...], 0.0)
    o = jnp.dot(h, w2_ref[...], preferred_element_type=jnp.float32)
    o_ref[...] = jnp.maximum(o + b2_ref[...], 0.0)


def _fuse_table(table, W1, b1, W2, b2):
    grid = (VOCAB // _BLK,)
    return pl.pallas_call(
        _mlp_body,
        grid=grid,
        in_specs=[
            pl.BlockSpec((_BLK, EMB), lambda i: (i, 0)),
            pl.BlockSpec((EMB, D1), lambda i: (0, 0)),
            pl.BlockSpec((1, D1), lambda i: (0, 0)),
            pl.BlockSpec((D1, D2), lambda i: (0, 0)),
            pl.BlockSpec((1, D2), lambda i: (0, 0)),
        ],
        out_specs=pl.BlockSpec((_BLK, D2), lambda i: (i, 0)),
        out_shape=jax.ShapeDtypeStruct((VOCAB, D2), jnp.float32),
    )(table, W1, b1.reshape(1, D1), W2, b2.reshape(1, D2))


_NC = 2
_NS = 16
_NW = _NC * _NS
_CH = 128
_NB = 5
_PF = 3


def _make_gather(n_tokens):
    assert n_tokens % (_NW * _CH) == 0
    n_ch = n_tokens // (_NW * _CH)
    b_per_w = n_ch * _CH
    mesh = plsc.VectorSubcoreMesh(core_axis_name="c", subcore_axis_name="s")

    @functools.partial(
        pl.kernel,
        out_type=jax.ShapeDtypeStruct((n_tokens, D2), jnp.float32),
        mesh=mesh,
        scratch_types=[
            pltpu.VMEM((n_ch, _CH), jnp.int32),
            pltpu.VMEM((_NB, _CH, D2), jnp.float32),
        ]
        + [pltpu.SemaphoreType.DMA] * (2 * _NB),
    )
    def gather_k(f_hbm, idx_hbm, out_hbm, idx_v, rows_v, *sems):
        gsem = sems[:_NB]
        ssem = sems[_NB:]
        wid = lax.axis_index("s") * _NC + lax.axis_index("c")
        pltpu.sync_copy(idx_hbm.at[wid], idx_v)
        base = wid * b_per_w

        def start_gather(j, b):
            pltpu.make_async_copy(f_hbm.at[idx_v.at[j]], rows_v.at[b], gsem[b]).start()

        def wait_gather(b):
            pltpu.make_async_copy(f_hbm.at[idx_v.at[0]], rows_v.at[b], gsem[b]).wait()

        def start_scatter(j, b):
            pltpu.make_async_copy(
                rows_v.at[b], out_hbm.at[pl.ds(base + j * _CH, _CH)], ssem[b]
            ).start()

        def wait_scatter(b):
            pltpu.make_async_copy(
                rows_v.at[b], out_hbm.at[pl.ds(base, _CH)], ssem[b]
            ).wait()

        for p in range(_PF):
            start_gather(p, p)

        @pl.loop(0, n_ch, step=_NB)
        def _(j):
            for b in range(_NB):
                jj = j + b
                wait_gather(b)
                start_scatter(jj, b)
                nxt = jj + _PF
                bn = (b + _PF) % _NB

                @pl.when(nxt < n_ch)
                def _():
                    @pl.when(nxt >= _NB)
                    def _():
                        wait_scatter(bn)

                    start_gather(nxt, bn)

        for b in range(_NB):
            wait_scatter(b)

    return gather_k


_gather_tokens = _make_gather(1024 * 200)


def kernel(inputs, table, W1, b1, W2, b2, is_training):
    fused = _fuse_table(table, W1, b1, W2, b2)
    b, t = inputs.shape
    idx3d = inputs.astype(jnp.int32).reshape(_NW, b * t // (_NW * _CH), _CH)
    out = _gather_tokens(fused, idx3d)
    return out.reshape(b, t, D2)
